# Initial kernel scaffold; baseline (speedup 1.0000x reference)
import jax
import jax.numpy as jnp
from jax.experimental import pallas as pl


def kernel(x, cluster, batch, pre_W0, pre_b0, pre_W1, pre_b1, l1_W0, l1_b0, l1_g, l1_b, l1_W1, l1_b1, l2_W0, l2_b0, l2_g, l2_b, l2_W1, l2_b1, norm_g, norm_b):
    raise NotImplementedError("write your pallas kernel here")


# speedup vs baseline: 2.2571x; 2.2571x over previous
import functools

import jax
import jax.numpy as jnp
from jax import lax
from jax.experimental import pallas as pl
from jax.experimental.pallas import tpu as pltpu
from jax.experimental.pallas import tpu_sc as plsc

N = 100000
S = 6250
D_IN = 128
H = 128
F = 64
FV = F // 16

NC = 2
NS = 16
NW = NC * NS
CT = 208
CH = 256

BLKA = 512
N_PAD = ((N + CH + BLKA - 1) // BLKA) * BLKA


def _silu(x):
    return x * jax.nn.sigmoid(x)


def _ln(u, g, b):
    m = jnp.mean(u, axis=1, keepdims=True)
    v = jnp.mean((u - m) ** 2, axis=1, keepdims=True)
    return (u - m) / jnp.sqrt(v + 1e-5) * g + b


def _mlp1_body(x_ref, w00, b00, w01, b01, w10, b10, g1, bb1, w11, b11, h_ref):
    t = _silu(jnp.dot(x_ref[...], w00[...], preferred_element_type=jnp.float32) + b00[...])
    o = jnp.dot(t, w01[...], preferred_element_type=jnp.float32) + b01[...]
    u = jnp.dot(o, w10[...], preferred_element_type=jnp.float32) + b10[...]
    u = _silu(_ln(u, g1[...], bb1[...]))
    h_ref[...] = jnp.dot(u, w11[...], preferred_element_type=jnp.float32) + b11[...]


def _mlp2_body(h_ref, e_ref, wa, wb, b20, g2, bb2, w21, b21, h2_ref):
    u = (jnp.dot(h_ref[...], wa[...], preferred_element_type=jnp.float32)
         + jnp.dot(e_ref[...], wb[...], preferred_element_type=jnp.float32) + b20[...])
    u = _silu(_ln(u, g2[...], bb2[...]))
    h2_ref[...] = jnp.dot(u, w21[...], preferred_element_type=jnp.float32) + b21[...]


def _final_body(a_ref, g_ref, b_ref, o_ref):
    a = a_ref[...]
    m = jnp.mean(a, axis=1, keepdims=True)
    v = jnp.mean((a - m) ** 2, axis=1, keepdims=True)
    nrm = (a - m) / jnp.sqrt(v + 1e-5)
    g = g_ref[...]
    b = b_ref[...]
    o_ref[...] = jnp.concatenate(
        [nrm * g[:, :F] + b[:, :F], nrm * g[:, F:] + b[:, F:]], axis=1)


def _row2(a):
    return a.reshape(1, -1)


def _wspec(shape):
    return pl.BlockSpec(shape, lambda i: (0, 0))


_MESH = plsc.VectorSubcoreMesh(core_axis_name="c", subcore_axis_name="s")
G = 16
NG = CH // G


def _m8(i):
    return pl.multiple_of(i, 8)


def _init_aggr(aggr_v):
    neg = jnp.full((16,), -jnp.inf, jnp.float32)

    def body(i, c):
        for j in range(FV):
            aggr_v[pl.ds(i * F + j * 16, 16)] = neg
        return c

    lax.fori_loop(0, CT, body, 0)


def _fix_neginf(aggr_v):
    def body(i, c):
        for j in range(FV):
            v = aggr_v[pl.ds(i * F + j * 16, 16)]
            aggr_v[pl.ds(i * F + j * 16, 16)] = jnp.where(v == -jnp.inf, 0.0, v)
        return c

    lax.fori_loop(0, CT, body, 0)


def _load_bounds(rs_hbm, rs_vm, w):
    pltpu.sync_copy(rs_hbm.at[pl.ds(_m8(w * 16), 16)], rs_vm)
    v = rs_vm[...]
    return v[0], v[1]


def _accum_group(cl_vm, h_v, aggr_v, g, lo, hi, ct0, extra=None):
    clv = cl_vm[pl.ds(g * G, G)]
    base = g * G
    neg = jnp.float32(-jnp.inf)
    for k in range(G):
        r = base + k
        valid = (r >= lo) & (r < hi)
        ci = jnp.minimum(jnp.maximum(clv[k] - ct0, 0), CT - 1)
        for j in range(FV):
            cur = aggr_v[pl.ds(ci * F + j * 16, 16)]
            hv = h_v[pl.ds(r * F + j * 16, 16)]
            hv = jnp.where(valid, hv, neg)
            aggr_v[pl.ds(ci * F + j * 16, 16)] = jnp.maximum(cur, hv)
    if extra is not None:
        bt_vm, pb_vm = extra
        nextv = cl_vm[pl.ds(g * G + 1, G)]
        btv = bt_vm[pl.ds(g * G, G)]
        rvec = base + lax.iota(jnp.int32, 16)
        vmask = (rvec >= lo) & (rvec < hi)
        emask = (clv != nextv) & vmask
        civ = jnp.minimum(jnp.maximum(clv - ct0, 0), CT - 1)
        plsc.store_scatter(pb_vm, [civ], btv, mask=emask)


def _group_range(lo, hi):
    return lo // G, (hi + G - 1) // G


@functools.partial(
    pl.kernel,
    out_type=jax.ShapeDtypeStruct((N_PAD * F,), jnp.float32),
    mesh=_MESH,
    compiler_params=pltpu.CompilerParams(needs_layout_passes=False),
    scratch_types=[
        pltpu.VMEM((16,), jnp.int32),
        pltpu.VMEM((CT * F,), jnp.float32),
        pltpu.VMEM((CH * F,), jnp.float32),
        pltpu.VMEM((CH + G,), jnp.int32),
        pltpu.VMEM((CH * F,), jnp.float32),
    ],
)
def _segmax_expand(h_hbm, cl_hbm, rs_hbm, e_hbm,
                   rs_vm, aggr_v, h_v, cl_vm, e_v):
    w = lax.axis_index("s") * NC + lax.axis_index("c")
    row_start, row_end = _load_bounds(rs_hbm, rs_vm, w)
    ct0 = w * CT

    _init_aggr(aggr_v)

    start0 = (row_start // 8) * 8
    nch = (row_end - start0 + CH - 1) // CH

    def chunk(k, c):
        start = start0 + k * CH
        pltpu.sync_copy(h_hbm.at[pl.ds(_m8(start * F), CH * F)], h_v)
        pltpu.sync_copy(cl_hbm.at[pl.ds(_m8(start), CH + G)], cl_vm)
        lo = jnp.maximum(row_start - start, 0)
        hi = jnp.minimum(row_end - start, CH)
        g0, g1 = _group_range(lo, hi)

        def grp(g, cc):
            _accum_group(cl_vm, h_v, aggr_v, g, lo, hi, ct0)
            return cc

        lax.fori_loop(g0, g1, grp, 0)
        return c

    lax.fori_loop(0, nch, chunk, 0)
    _fix_neginf(aggr_v)

    def chunk2(k, c):
        start = start0 + k * CH
        pltpu.sync_copy(cl_hbm.at[pl.ds(_m8(start), CH + G)], cl_vm)
        lo = jnp.maximum(row_start - start, 0)
        hi = jnp.minimum(row_end - start, CH)
        g0, g1 = _group_range(lo, hi)

        def grp(g, cc):
            clv = cl_vm[pl.ds(g * G, G)]
            base = g * G
            for k16 in range(G):
                r = base + k16
                ci = jnp.minimum(jnp.maximum(clv[k16] - ct0, 0), CT - 1)
                for j in range(FV):
                    e_v[pl.ds(r * F + j * 16, 16)] = aggr_v[pl.ds(ci * F + j * 16, 16)]
            return cc

        lax.fori_loop(g0, g1, grp, 0)

        rem = hi - lo

        @pl.when(rem == CH)
        def _():
            pltpu.sync_copy(e_v, e_hbm.at[pl.ds(_m8(start * F), CH * F)])

        @pl.when((rem > 0) & (rem < CH))
        def _():
            for sz in (128, 64, 32, 16, 8, 4, 2, 1):
                off = lo + (rem // (2 * sz)) * (2 * sz)

                @pl.when((rem // sz) % 2 == 1)
                def _(off=off, sz=sz):
                    pltpu.sync_copy(
                        e_v.at[pl.ds(off * F, sz * F)],
                        e_hbm.at[pl.ds(_m8((start + off) * F), sz * F)])
        return c

    lax.fori_loop(0, nch, chunk2, 0)


@functools.partial(
    pl.kernel,
    out_type=[jax.ShapeDtypeStruct((NW * CT * F,), jnp.float32),
              jax.ShapeDtypeStruct((NW * CT,), jnp.int32)],
    mesh=_MESH,
    compiler_params=pltpu.CompilerParams(needs_layout_passes=False),
    scratch_types=[
        pltpu.VMEM((16,), jnp.int32),
        pltpu.VMEM((CT * F,), jnp.float32),
        pltpu.VMEM((CH * F,), jnp.float32),
        pltpu.VMEM((CH + G,), jnp.int32),
        pltpu.VMEM((CH + G,), jnp.int32),
        pltpu.VMEM((CT,), jnp.int32),
    ],
)
def _segmax_batch(h_hbm, cl_hbm, bt_hbm, rs_hbm, aggr_hbm, pb_hbm,
                  rs_vm, aggr_v, h_v, cl_vm, bt_vm, pb_vm):
    w = lax.axis_index("s") * NC + lax.axis_index("c")
    row_start, row_end = _load_bounds(rs_hbm, rs_vm, w)
    ct0 = w * CT

    _init_aggr(aggr_v)
    zero16 = jnp.zeros((16,), jnp.int32)

    def initpb(i, c):
        pb_vm[pl.ds(i * 16, 16)] = zero16
        return c

    lax.fori_loop(0, CT // 16, initpb, 0)

    start0 = (row_start // 8) * 8
    nch = (row_end - start0 + CH - 1) // CH

    def chunk(k, c):
        start = start0 + k * CH
        pltpu.sync_copy(h_hbm.at[pl.ds(_m8(start * F), CH * F)], h_v)
        pltpu.sync_copy(cl_hbm.at[pl.ds(_m8(start), CH + G)], cl_vm)
        pltpu.sync_copy(bt_hbm.at[pl.ds(_m8(start), CH + G)], bt_vm)
        lo = jnp.maximum(row_start - start, 0)
        hi = jnp.minimum(row_end - start, CH)
        g0, g1 = _group_range(lo, hi)

        def grp(g, cc):
            _accum_group(cl_vm, h_v, aggr_v, g, lo, hi, ct0, extra=(bt_vm, pb_vm))
            return cc

        lax.fori_loop(g0, g1, grp, 0)
        return c

    lax.fori_loop(0, nch, chunk, 0)

    _fix_neginf(aggr_v)
    pltpu.sync_copy(aggr_v, aggr_hbm.at[pl.ds(_m8(ct0 * F), CT * F)])
    pltpu.sync_copy(pb_vm, pb_hbm.at[pl.ds(_m8(ct0), CT)])


def kernel(x, cluster, batch, pre_W0, pre_b0, pre_W1, pre_b1,
           l1_W0, l1_b0, l1_g, l1_b, l1_W1, l1_b1,
           l2_W0, l2_b0, l2_g, l2_b, l2_W1, l2_b1,
           norm_g, norm_b):
    grid_a = (N_PAD // BLKA,)

    h1 = pl.pallas_call(
        _mlp1_body,
        grid=grid_a,
        in_specs=[
            pl.BlockSpec((BLKA, D_IN), lambda i: (i, 0)),
            _wspec((D_IN, H)), _wspec((1, H)),
            _wspec((H, H)), _wspec((1, H)),
            _wspec((H, H)), _wspec((1, H)),
            _wspec((1, H)), _wspec((1, H)),
            _wspec((H, F)), _wspec((1, F)),
        ],
        out_specs=pl.BlockSpec((BLKA, F), lambda i: (i, 0)),
        out_shape=jax.ShapeDtypeStruct((N_PAD, F), jnp.float32),
    )(x, pre_W0, _row2(pre_b0), pre_W1, _row2(pre_b1),
      l1_W0, _row2(l1_b0), _row2(l1_g), _row2(l1_b), l1_W1, _row2(l1_b1))

    bounds = jnp.arange(NW + 1, dtype=jnp.int32) * CT
    rs = jnp.searchsorted(cluster, bounds, side="left").astype(jnp.int32)
    rs2 = jnp.zeros((NW, 16), jnp.int32)
    rs2 = rs2.at[:, 0].set(rs[:NW]).at[:, 1].set(rs[1:NW + 1]).reshape(-1)
    cl_pad = jnp.pad(cluster, (0, N_PAD - N), constant_values=S)
    bt_pad = jnp.pad(batch, (0, N_PAD - N))

    e1 = _segmax_expand(h1.reshape(-1), cl_pad, rs2).reshape(N_PAD, F)

    wa = l2_W0[:F, :]
    wb = l2_W0[F:, :]
    h2 = pl.pallas_call(
        _mlp2_body,
        grid=grid_a,
        in_specs=[
            pl.BlockSpec((BLKA, F), lambda i: (i, 0)),
            pl.BlockSpec((BLKA, F), lambda i: (i, 0)),
            _wspec((F, H)), _wspec((F, H)), _wspec((1, H)),
            _wspec((1, H)), _wspec((1, H)),
            _wspec((H, F)), _wspec((1, F)),
        ],
        out_specs=pl.BlockSpec((BLKA, F), lambda i: (i, 0)),
        out_shape=jax.ShapeDtypeStruct((N_PAD, F), jnp.float32),
    )(h1, e1, wa, wb, _row2(l2_b0), _row2(l2_g), _row2(l2_b),
      l2_W1, _row2(l2_b1))

    aggr2, pb = _segmax_batch(h2.reshape(-1), cl_pad, bt_pad, rs2)
    aggr2 = aggr2.reshape(NW * CT, F)

    BLKE = 512
    grid_e = ((NW * CT) // BLKE,)
    out = pl.pallas_call(
        _final_body,
        grid=grid_e,
        in_specs=[
            pl.BlockSpec((BLKE, F), lambda i: (i, 0)),
            _wspec((1, H)), _wspec((1, H)),
        ],
        out_specs=pl.BlockSpec((BLKE, H), lambda i: (i, 0)),
        out_shape=jax.ShapeDtypeStruct((S, H), jnp.float32),
    )(aggr2, _row2(norm_g), _row2(norm_b))

    return out, pb[:S]


# speedup vs baseline: 3.0416x; 1.3476x over previous
import functools

import jax
import jax.numpy as jnp
from jax import lax
from jax.experimental import pallas as pl
from jax.experimental.pallas import tpu as pltpu
from jax.experimental.pallas import tpu_sc as plsc

N = 100000
S = 6250
D_IN = 128
H = 128
F = 64
FV = F // 16

NC = 2
NS = 16
NW = NC * NS
CT = 208
CH = 256

BLKA = 1024
N_PAD = ((N + CH + BLKA - 1) // BLKA) * BLKA


def _silu(x):
    return x * jax.nn.sigmoid(x)


def _ln(u, g, b):
    m = jnp.mean(u, axis=1, keepdims=True)
    ex2 = jnp.mean(u * u, axis=1, keepdims=True)
    v = ex2 - m * m
    return (u - m) * lax.rsqrt(v + 1e-5) * g + b


def _bdot(a, b):
    return jnp.dot(a.astype(jnp.bfloat16), b, preferred_element_type=jnp.float32)


def _mlp1_body(x_ref, w00, b00, w01, b01, w10, b10, g1, bb1, w11, b11, h_ref):
    t = _silu(_bdot(x_ref[...], w00[...]) + b00[...])
    o = _bdot(t, w01[...]) + b01[...]
    u = _bdot(o, w10[...]) + b10[...]
    u = _silu(_ln(u, g1[...], bb1[...]))
    h_ref[...] = _bdot(u, w11[...]) + b11[...]


def _mlp2_body(h_ref, e_ref, wa, wb, b20, g2, bb2, w21, b21, h2_ref):
    u = _bdot(h_ref[...], wa[...]) + _bdot(e_ref[...], wb[...]) + b20[...]
    u = _silu(_ln(u, g2[...], bb2[...]))
    h2_ref[...] = _bdot(u, w21[...]) + b21[...]


def _final_body(a_ref, g_ref, b_ref, o_ref):
    a = a_ref[...]
    m = jnp.mean(a, axis=1, keepdims=True)
    v = jnp.mean((a - m) ** 2, axis=1, keepdims=True)
    nrm = (a - m) * lax.rsqrt(v + 1e-5)
    g = g_ref[...]
    b = b_ref[...]
    o_ref[...] = jnp.concatenate(
        [nrm * g[:, :F] + b[:, :F], nrm * g[:, F:] + b[:, F:]], axis=1)


def _row2(a):
    return a.reshape(1, -1)


def _wspec(shape):
    return pl.BlockSpec(shape, lambda i: (0, 0))


_MESH = plsc.VectorSubcoreMesh(core_axis_name="c", subcore_axis_name="s")
G = 16
NG = CH // G


def _m8(i):
    return pl.multiple_of(i, 8)


def _init_aggr(aggr_v):
    neg = jnp.full((16,), -jnp.inf, jnp.float32)

    def body(i, c):
        for j in range(FV):
            aggr_v[pl.ds(i * F + j * 16, 16)] = neg
        return c

    lax.fori_loop(0, CT, body, 0)


def _fix_neginf(aggr_v):
    def body(i, c):
        for j in range(FV):
            v = aggr_v[pl.ds(i * F + j * 16, 16)]
            aggr_v[pl.ds(i * F + j * 16, 16)] = jnp.where(v == -jnp.inf, 0.0, v)
        return c

    lax.fori_loop(0, CT, body, 0)


def _load_bounds(rs_hbm, rs_vm, w):
    pltpu.sync_copy(rs_hbm.at[pl.ds(_m8(w * 16), 16)], rs_vm)
    v = rs_vm[...]
    return v[0], v[1]


def _accum_group(cl_vm, h_v, aggr_v, g, lo, hi, ct0, extra=None):
    clv = cl_vm[pl.ds(g * G, G)]
    base = g * G
    neg = jnp.float32(-jnp.inf)
    for k in range(G):
        r = base + k
        valid = (r >= lo) & (r < hi)
        ci = jnp.minimum(jnp.maximum(clv[k] - ct0, 0), CT - 1)
        for j in range(FV):
            cur = aggr_v[pl.ds(ci * F + j * 16, 16)]
            hv = h_v[pl.ds(r * F + j * 16, 16)]
            hv = jnp.where(valid, hv, neg)
            aggr_v[pl.ds(ci * F + j * 16, 16)] = jnp.maximum(cur, hv)
    if extra is not None:
        bt_vm, pb_vm = extra
        nextv = cl_vm[pl.ds(g * G + 1, G)]
        btv = bt_vm[pl.ds(g * G, G)]
        rvec = base + lax.iota(jnp.int32, 16)
        vmask = (rvec >= lo) & (rvec < hi)
        emask = (clv != nextv) & vmask
        civ = jnp.minimum(jnp.maximum(clv - ct0, 0), CT - 1)
        plsc.store_scatter(pb_vm, [civ], btv, mask=emask)


def _group_range(lo, hi):
    return lo // G, (hi + G - 1) // G


@functools.partial(
    pl.kernel,
    out_type=jax.ShapeDtypeStruct((N_PAD * F,), jnp.float32),
    mesh=_MESH,
    compiler_params=pltpu.CompilerParams(needs_layout_passes=False),
    scratch_types=[
        pltpu.VMEM((16,), jnp.int32),
        pltpu.VMEM((CT * F,), jnp.float32),
        pltpu.VMEM((CH * F,), jnp.float32),
        pltpu.VMEM((CH * F,), jnp.float32),
        pltpu.VMEM((CH + G,), jnp.int32),
        pltpu.VMEM((CH + G,), jnp.int32),
        pltpu.SemaphoreType.DMA,
        pltpu.SemaphoreType.DMA,
        pltpu.SemaphoreType.DMA,
        pltpu.SemaphoreType.DMA,
        pltpu.SemaphoreType.DMA,
        pltpu.SemaphoreType.DMA,
    ],
)
def _segmax_expand(h_hbm, cl_hbm, rs_hbm, e_hbm,
                   rs_vm, aggr_v, h_v0, h_v1, cl_v0, cl_v1,
                   sh0, sh1, sc0, sc1, so0, so1):
    w = lax.axis_index("s") * NC + lax.axis_index("c")
    row_start, row_end = _load_bounds(rs_hbm, rs_vm, w)
    ct0 = w * CT
    _init_aggr(aggr_v)

    start0 = (row_start // 8) * 8
    nch = (row_end - start0 + CH - 1) // CH

    def hslice(k):
        return h_hbm.at[pl.ds(_m8((start0 + k * CH) * F), CH * F)]

    def clslice(k):
        return cl_hbm.at[pl.ds(_m8(start0 + k * CH), CH + G)]

    def lohi(k):
        start = start0 + k * CH
        return (jnp.maximum(row_start - start, 0),
                jnp.minimum(row_end - start, CH))

    @pl.when(nch > 0)
    def _():
        pltpu.async_copy(hslice(0), h_v0, sh0)
        pltpu.async_copy(clslice(0), cl_v0, sc0)

    def chunkA(k, c):
        @pl.when(k + 1 < nch)
        def _():
            @pl.when(k % 2 == 0)
            def _():
                pltpu.async_copy(hslice(k + 1), h_v1, sh1)
                pltpu.async_copy(clslice(k + 1), cl_v1, sc1)

            @pl.when(k % 2 == 1)
            def _():
                pltpu.async_copy(hslice(k + 1), h_v0, sh0)
                pltpu.async_copy(clslice(k + 1), cl_v0, sc0)

        lo, hi = lohi(k)
        g0, g1 = _group_range(lo, hi)

        def proc(h_v, cl_vm):
            def grp(g, cc):
                _accum_group(cl_vm, h_v, aggr_v, g, lo, hi, ct0)
                return cc

            lax.fori_loop(g0, g1, grp, 0)

        @pl.when(k % 2 == 0)
        def _():
            pltpu.make_async_copy(hslice(k), h_v0, sh0).wait()
            pltpu.make_async_copy(clslice(k), cl_v0, sc0).wait()
            proc(h_v0, cl_v0)

        @pl.when(k % 2 == 1)
        def _():
            pltpu.make_async_copy(hslice(k), h_v1, sh1).wait()
            pltpu.make_async_copy(clslice(k), cl_v1, sc1).wait()
            proc(h_v1, cl_v1)

        return c

    lax.fori_loop(0, nch, chunkA, 0)
    _fix_neginf(aggr_v)

    def full(k):
        lo, hi = lohi(k)
        return (lo == 0) & (hi == CH)

    @pl.when(nch > 0)
    def _():
        pltpu.async_copy(clslice(0), cl_v0, sc0)

    def chunkB(k, c):
        @pl.when(k + 1 < nch)
        def _():
            @pl.when(k % 2 == 0)
            def _():
                pltpu.async_copy(clslice(k + 1), cl_v1, sc1)

            @pl.when(k % 2 == 1)
            def _():
                pltpu.async_copy(clslice(k + 1), cl_v0, sc0)

        start = start0 + k * CH
        lo, hi = lohi(k)
        g0, g1 = _group_range(lo, hi)

        def proc(cl_vm, e_v, so):
            @pl.when((k >= 2) & full(k - 2))
            def _():
                pltpu.make_async_copy(
                    e_v,
                    e_hbm.at[pl.ds(_m8((start0 + (k - 2) * CH) * F), CH * F)],
                    so).wait()

            def grp(g, cc):
                clv = cl_vm[pl.ds(g * G, G)]
                base = g * G
                for k16 in range(G):
                    r = base + k16
                    ci = jnp.minimum(jnp.maximum(clv[k16] - ct0, 0), CT - 1)
                    for j in range(FV):
                        e_v[pl.ds(r * F + j * 16, 16)] = aggr_v[pl.ds(ci * F + j * 16, 16)]
                return cc

            lax.fori_loop(g0, g1, grp, 0)
            rem = hi - lo

            @pl.when(rem == CH)
            def _():
                pltpu.async_copy(e_v, e_hbm.at[pl.ds(_m8(start * F), CH * F)], so)

            @pl.when((rem > 0) & (rem < CH))
            def _():
                for sz in (256, 128, 64, 32, 16, 8, 4, 2, 1):
                    off = lo + (rem // (2 * sz)) * (2 * sz)

                    @pl.when((rem // sz) % 2 == 1)
                    def _(off=off, sz=sz):
                        pltpu.sync_copy(
                            e_v.at[pl.ds(off * F, sz * F)],
                            e_hbm.at[pl.ds(_m8((start + off) * F), sz * F)])

        @pl.when(k % 2 == 0)
        def _():
            pltpu.make_async_copy(clslice(k), cl_v0, sc0).wait()
            proc(cl_v0, h_v0, so0)

        @pl.when(k % 2 == 1)
        def _():
            pltpu.make_async_copy(clslice(k), cl_v1, sc1).wait()
            proc(cl_v1, h_v1, so1)

        return c

    lax.fori_loop(0, nch, chunkB, 0)

    for d in (1, 2):
        kd = nch - d

        @pl.when((kd >= 0) & full(kd))
        def _(kd=kd):
            @pl.when(kd % 2 == 0)
            def _():
                pltpu.make_async_copy(
                    h_v0, e_hbm.at[pl.ds(_m8((start0 + kd * CH) * F), CH * F)],
                    so0).wait()

            @pl.when(kd % 2 == 1)
            def _():
                pltpu.make_async_copy(
                    h_v1, e_hbm.at[pl.ds(_m8((start0 + kd * CH) * F), CH * F)],
                    so1).wait()


@functools.partial(
    pl.kernel,
    out_type=[jax.ShapeDtypeStruct((NW * CT * F,), jnp.float32),
              jax.ShapeDtypeStruct((NW * CT,), jnp.int32)],
    mesh=_MESH,
    compiler_params=pltpu.CompilerParams(needs_layout_passes=False),
    scratch_types=[
        pltpu.VMEM((16,), jnp.int32),
        pltpu.VMEM((CT * F,), jnp.float32),
        pltpu.VMEM((CH * F,), jnp.float32),
        pltpu.VMEM((CH * F,), jnp.float32),
        pltpu.VMEM((CH + G,), jnp.int32),
        pltpu.VMEM((CH + G,), jnp.int32),
        pltpu.VMEM((CH + G,), jnp.int32),
        pltpu.VMEM((CH + G,), jnp.int32),
        pltpu.VMEM((CT,), jnp.int32),
        pltpu.SemaphoreType.DMA,
        pltpu.SemaphoreType.DMA,
        pltpu.SemaphoreType.DMA,
        pltpu.SemaphoreType.DMA,
        pltpu.SemaphoreType.DMA,
        pltpu.SemaphoreType.DMA,
    ],
)
def _segmax_batch(h_hbm, cl_hbm, bt_hbm, rs_hbm, aggr_hbm, pb_hbm,
                  rs_vm, aggr_v, h_v0, h_v1, cl_v0, cl_v1, bt_v0, bt_v1,
                  pb_vm, sh0, sh1, sc0, sc1, sb0, sb1):
    w = lax.axis_index("s") * NC + lax.axis_index("c")
    row_start, row_end = _load_bounds(rs_hbm, rs_vm, w)
    ct0 = w * CT

    _init_aggr(aggr_v)
    zero16 = jnp.zeros((16,), jnp.int32)

    def initpb(i, c):
        pb_vm[pl.ds(i * 16, 16)] = zero16
        return c

    lax.fori_loop(0, CT // 16, initpb, 0)

    start0 = (row_start // 8) * 8
    nch = (row_end - start0 + CH - 1) // CH

    def hslice(k):
        return h_hbm.at[pl.ds(_m8((start0 + k * CH) * F), CH * F)]

    def clslice(k):
        return cl_hbm.at[pl.ds(_m8(start0 + k * CH), CH + G)]

    def btslice(k):
        return bt_hbm.at[pl.ds(_m8(start0 + k * CH), CH + G)]

    @pl.when(nch > 0)
    def _():
        pltpu.async_copy(hslice(0), h_v0, sh0)
        pltpu.async_copy(clslice(0), cl_v0, sc0)
        pltpu.async_copy(btslice(0), bt_v0, sb0)

    def chunk(k, c):
        @pl.when(k + 1 < nch)
        def _():
            @pl.when(k % 2 == 0)
            def _():
                pltpu.async_copy(hslice(k + 1), h_v1, sh1)
                pltpu.async_copy(clslice(k + 1), cl_v1, sc1)
                pltpu.async_copy(btslice(k + 1), bt_v1, sb1)

            @pl.when(k % 2 == 1)
            def _():
                pltpu.async_copy(hslice(k + 1), h_v0, sh0)
                pltpu.async_copy(clslice(k + 1), cl_v0, sc0)
                pltpu.async_copy(btslice(k + 1), bt_v0, sb0)

        start = start0 + k * CH
        lo = jnp.maximum(row_start - start, 0)
        hi = jnp.minimum(row_end - start, CH)
        g0, g1 = _group_range(lo, hi)

        def proc(h_v, cl_vm, bt_vm):
            def grp(g, cc):
                _accum_group(cl_vm, h_v, aggr_v, g, lo, hi, ct0,
                             extra=(bt_vm, pb_vm))
                return cc

            lax.fori_loop(g0, g1, grp, 0)

        @pl.when(k % 2 == 0)
        def _():
            pltpu.make_async_copy(hslice(k), h_v0, sh0).wait()
            pltpu.make_async_copy(clslice(k), cl_v0, sc0).wait()
            pltpu.make_async_copy(btslice(k), bt_v0, sb0).wait()
            proc(h_v0, cl_v0, bt_v0)

        @pl.when(k % 2 == 1)
        def _():
            pltpu.make_async_copy(hslice(k), h_v1, sh1).wait()
            pltpu.make_async_copy(clslice(k), cl_v1, sc1).wait()
            pltpu.make_async_copy(btslice(k), bt_v1, sb1).wait()
            proc(h_v1, cl_v1, bt_v1)

        return c

    lax.fori_loop(0, nch, chunk, 0)

    _fix_neginf(aggr_v)
    pltpu.sync_copy(aggr_v, aggr_hbm.at[pl.ds(_m8(ct0 * F), CT * F)])
    pltpu.sync_copy(pb_vm, pb_hbm.at[pl.ds(_m8(ct0), CT)])


def kernel(x, cluster, batch, pre_W0, pre_b0, pre_W1, pre_b1,
           l1_W0, l1_b0, l1_g, l1_b, l1_W1, l1_b1,
           l2_W0, l2_b0, l2_g, l2_b, l2_W1, l2_b1,
           norm_g, norm_b):
    grid_a = (N_PAD // BLKA,)

    h1 = pl.pallas_call(
        _mlp1_body,
        grid=grid_a,
        in_specs=[
            pl.BlockSpec((BLKA, D_IN), lambda i: (i, 0)),
            _wspec((D_IN, H)), _wspec((1, H)),
            _wspec((H, H)), _wspec((1, H)),
            _wspec((H, H)), _wspec((1, H)),
            _wspec((1, H)), _wspec((1, H)),
            _wspec((H, F)), _wspec((1, F)),
        ],
        out_specs=pl.BlockSpec((BLKA, F), lambda i: (i, 0)),
        out_shape=jax.ShapeDtypeStruct((N_PAD, F), jnp.float32),
    )(x, pre_W0.astype(jnp.bfloat16), _row2(pre_b0),
      pre_W1.astype(jnp.bfloat16), _row2(pre_b1),
      l1_W0.astype(jnp.bfloat16), _row2(l1_b0), _row2(l1_g), _row2(l1_b),
      l1_W1.astype(jnp.bfloat16), _row2(l1_b1))

    bounds = jnp.arange(NW + 1, dtype=jnp.int32) * CT
    rs = jnp.searchsorted(cluster, bounds, side="left").astype(jnp.int32)
    rs2 = jnp.zeros((NW, 16), jnp.int32)
    rs2 = rs2.at[:, 0].set(rs[:NW]).at[:, 1].set(rs[1:NW + 1]).reshape(-1)
    cl_pad = jnp.pad(cluster, (0, N_PAD - N), constant_values=S)
    bt_pad = jnp.pad(batch, (0, N_PAD - N))

    e1 = _segmax_expand(h1.reshape(-1), cl_pad, rs2).reshape(N_PAD, F)

    wa = l2_W0[:F, :].astype(jnp.bfloat16)
    wb = l2_W0[F:, :].astype(jnp.bfloat16)
    h2 = pl.pallas_call(
        _mlp2_body,
        grid=grid_a,
        in_specs=[
            pl.BlockSpec((BLKA, F), lambda i: (i, 0)),
            pl.BlockSpec((BLKA, F), lambda i: (i, 0)),
            _wspec((F, H)), _wspec((F, H)), _wspec((1, H)),
            _wspec((1, H)), _wspec((1, H)),
            _wspec((H, F)), _wspec((1, F)),
        ],
        out_specs=pl.BlockSpec((BLKA, F), lambda i: (i, 0)),
        out_shape=jax.ShapeDtypeStruct((N_PAD, F), jnp.float32),
    )(h1, e1, wa, wb, _row2(l2_b0), _row2(l2_g), _row2(l2_b),
      l2_W1.astype(jnp.bfloat16), _row2(l2_b1))

    aggr2, pb = _segmax_batch(h2.reshape(-1), cl_pad, bt_pad, rs2)
    aggr2 = aggr2.reshape(NW * CT, F)

    BLKE = 512
    grid_e = ((NW * CT) // BLKE,)
    out = pl.pallas_call(
        _final_body,
        grid=grid_e,
        in_specs=[
            pl.BlockSpec((BLKE, F), lambda i: (i, 0)),
            _wspec((1, H)), _wspec((1, H)),
        ],
        out_specs=pl.BlockSpec((BLKE, H), lambda i: (i, 0)),
        out_shape=jax.ShapeDtypeStruct((S, H), jnp.float32),
    )(aggr2, _row2(norm_g), _row2(norm_b))

    return out, pb[:S]


# speedup vs baseline: 3.6978x; 1.2158x over previous
import functools

import jax
import jax.numpy as jnp
from jax import lax
from jax.experimental import pallas as pl
from jax.experimental.pallas import tpu as pltpu
from jax.experimental.pallas import tpu_sc as plsc

N = 100000
S = 6250
D_IN = 128
H = 128
F = 64
FV = F // 16

NC = 2
NS = 16
NW = NC * NS
CT = 208
CH = 256

BLKA = 1024
N_PAD = ((N + CH + BLKA - 1) // BLKA) * BLKA


def _silu(x):
    return x * jax.nn.sigmoid(x)


def _ln(u, g, b):
    m = jnp.mean(u, axis=1, keepdims=True)
    ex2 = jnp.mean(u * u, axis=1, keepdims=True)
    v = ex2 - m * m
    return (u - m) * lax.rsqrt(v + 1e-5) * g + b


def _bdot(a, b):
    return jnp.dot(a.astype(jnp.bfloat16), b, preferred_element_type=jnp.float32)


def _mlp1_body(x_ref, w00, b00, w01, b01, w10, b10, g1, bb1, w11, b11, h_ref):
    t = _silu(_bdot(x_ref[...], w00[...]) + b00[...])
    o = _bdot(t, w01[...]) + b01[...]
    u = _bdot(o, w10[...]) + b10[...]
    u = _silu(_ln(u, g1[...], bb1[...]))
    h_ref[...] = _bdot(u, w11[...]) + b11[...]


def _mlp2_body(h_ref, e_ref, wa, wb, b20, g2, bb2, w21, b21, h2_ref):
    u = _bdot(h_ref[...], wa[...]) + _bdot(e_ref[...], wb[...]) + b20[...]
    ul = _silu(_ln(u[:, :H], g2[...], bb2[...]))
    ur = _silu(_ln(u[:, H:], g2[...], bb2[...]))
    t = jnp.concatenate([ul, ur], axis=1)
    h2_ref[...] = _bdot(t, w21[...]) + b21[...]


def _final_body(a_ref, g_ref, b_ref, o_ref):
    a = a_ref[...]
    m = jnp.mean(a, axis=1, keepdims=True)
    v = jnp.mean((a - m) ** 2, axis=1, keepdims=True)
    nrm = (a - m) * lax.rsqrt(v + 1e-5)
    g = g_ref[...]
    b = b_ref[...]
    o_ref[...] = jnp.concatenate(
        [nrm * g[:, :F] + b[:, :F], nrm * g[:, F:] + b[:, F:]], axis=1)


def _row2(a):
    return a.reshape(1, -1)


def _wspec(shape):
    return pl.BlockSpec(shape, lambda i: (0, 0))


_MESH = plsc.VectorSubcoreMesh(core_axis_name="c", subcore_axis_name="s")
G = 16
NG = CH // G


def _m8(i):
    return pl.multiple_of(i, 8)


def _init_aggr(aggr_v):
    neg = jnp.full((16,), -jnp.inf, jnp.float32)

    def body(i, c):
        for j in range(FV):
            aggr_v[pl.ds(i * F + j * 16, 16)] = neg
        return c

    lax.fori_loop(0, CT, body, 0)


def _fix_neginf(aggr_v):
    def body(i, c):
        for j in range(FV):
            v = aggr_v[pl.ds(i * F + j * 16, 16)]
            aggr_v[pl.ds(i * F + j * 16, 16)] = jnp.where(v == -jnp.inf, 0.0, v)
        return c

    lax.fori_loop(0, CT, body, 0)


def _load_bounds(rs_hbm, rs_vm, w):
    pltpu.sync_copy(rs_hbm.at[pl.ds(_m8(w * 16), 16)], rs_vm)
    v = rs_vm[...]
    return v[0], v[1]


def _accum_group(cl_vm, h_v, aggr_v, g, lo, hi, ct0, extra=None):
    clv = cl_vm[pl.ds(g * G, G)]
    base = g * G
    neg = jnp.float32(-jnp.inf)
    for k in range(G):
        r = base + k
        valid = (r >= lo) & (r < hi)
        ci = jnp.minimum(jnp.maximum(clv[k] - ct0, 0), CT - 1)
        for j in range(FV):
            cur = aggr_v[pl.ds(ci * F + j * 16, 16)]
            hv = h_v[pl.ds(r * F + j * 16, 16)]
            hv = jnp.where(valid, hv, neg)
            aggr_v[pl.ds(ci * F + j * 16, 16)] = jnp.maximum(cur, hv)
    if extra is not None:
        bt_vm, pb_vm = extra
        nextv = cl_vm[pl.ds(g * G + 1, G)]
        btv = bt_vm[pl.ds(g * G, G)]
        rvec = base + lax.iota(jnp.int32, 16)
        vmask = (rvec >= lo) & (rvec < hi)
        emask = (clv != nextv) & vmask
        civ = jnp.minimum(jnp.maximum(clv - ct0, 0), CT - 1)
        plsc.store_scatter(pb_vm, [civ], btv, mask=emask)


def _group_range(lo, hi):
    return lo // G, (hi + G - 1) // G


@functools.partial(
    pl.kernel,
    out_type=jax.ShapeDtypeStruct((N_PAD * F,), jnp.float32),
    mesh=_MESH,
    compiler_params=pltpu.CompilerParams(needs_layout_passes=False),
    scratch_types=[
        pltpu.VMEM((16,), jnp.int32),
        pltpu.VMEM((CT * F,), jnp.float32),
        pltpu.VMEM((CH * F,), jnp.float32),
        pltpu.VMEM((CH * F,), jnp.float32),
        pltpu.VMEM((CH + G,), jnp.int32),
        pltpu.VMEM((CH + G,), jnp.int32),
        pltpu.SemaphoreType.DMA,
        pltpu.SemaphoreType.DMA,
        pltpu.SemaphoreType.DMA,
        pltpu.SemaphoreType.DMA,
        pltpu.SemaphoreType.DMA,
        pltpu.SemaphoreType.DMA,
    ],
)
def _segmax_expand(h_hbm, cl_hbm, rs_hbm, e_hbm,
                   rs_vm, aggr_v, h_v0, h_v1, cl_v0, cl_v1,
                   sh0, sh1, sc0, sc1, so0, so1):
    w = lax.axis_index("s") * NC + lax.axis_index("c")
    row_start, row_end = _load_bounds(rs_hbm, rs_vm, w)
    ct0 = w * CT
    _init_aggr(aggr_v)

    start0 = (row_start // 8) * 8
    nch = (row_end - start0 + CH - 1) // CH

    def hslice(k):
        return h_hbm.at[pl.ds(_m8((start0 + k * CH) * F), CH * F)]

    def clslice(k):
        return cl_hbm.at[pl.ds(_m8(start0 + k * CH), CH + G)]

    def lohi(k):
        start = start0 + k * CH
        return (jnp.maximum(row_start - start, 0),
                jnp.minimum(row_end - start, CH))

    @pl.when(nch > 0)
    def _():
        pltpu.async_copy(hslice(0), h_v0, sh0)
        pltpu.async_copy(clslice(0), cl_v0, sc0)

    def chunkA(k, c):
        @pl.when(k + 1 < nch)
        def _():
            @pl.when(k % 2 == 0)
            def _():
                pltpu.async_copy(hslice(k + 1), h_v1, sh1)
                pltpu.async_copy(clslice(k + 1), cl_v1, sc1)

            @pl.when(k % 2 == 1)
            def _():
                pltpu.async_copy(hslice(k + 1), h_v0, sh0)
                pltpu.async_copy(clslice(k + 1), cl_v0, sc0)

        lo, hi = lohi(k)
        g0, g1 = _group_range(lo, hi)

        def proc(h_v, cl_vm):
            def grp(g, cc):
                _accum_group(cl_vm, h_v, aggr_v, g, lo, hi, ct0)
                return cc

            lax.fori_loop(g0, g1, grp, 0)

        @pl.when(k % 2 == 0)
        def _():
            pltpu.make_async_copy(hslice(k), h_v0, sh0).wait()
            pltpu.make_async_copy(clslice(k), cl_v0, sc0).wait()
            proc(h_v0, cl_v0)

        @pl.when(k % 2 == 1)
        def _():
            pltpu.make_async_copy(hslice(k), h_v1, sh1).wait()
            pltpu.make_async_copy(clslice(k), cl_v1, sc1).wait()
            proc(h_v1, cl_v1)

        return c

    lax.fori_loop(0, nch, chunkA, 0)
    _fix_neginf(aggr_v)

    def full(k):
        lo, hi = lohi(k)
        return (lo == 0) & (hi == CH)

    @pl.when(nch > 0)
    def _():
        pltpu.async_copy(clslice(0), cl_v0, sc0)

    def chunkB(k, c):
        @pl.when(k + 1 < nch)
        def _():
            @pl.when(k % 2 == 0)
            def _():
                pltpu.async_copy(clslice(k + 1), cl_v1, sc1)

            @pl.when(k % 2 == 1)
            def _():
                pltpu.async_copy(clslice(k + 1), cl_v0, sc0)

        start = start0 + k * CH
        lo, hi = lohi(k)
        g0, g1 = _group_range(lo, hi)

        def proc(cl_vm, e_v, so):
            @pl.when((k >= 2) & full(k - 2))
            def _():
                pltpu.make_async_copy(
                    e_v,
                    e_hbm.at[pl.ds(_m8((start0 + (k - 2) * CH) * F), CH * F)],
                    so).wait()

            def grp(g, cc):
                clv = cl_vm[pl.ds(g * G, G)]
                base = g * G
                for k16 in range(G):
                    r = base + k16
                    ci = jnp.minimum(jnp.maximum(clv[k16] - ct0, 0), CT - 1)
                    for j in range(FV):
                        e_v[pl.ds(r * F + j * 16, 16)] = aggr_v[pl.ds(ci * F + j * 16, 16)]
                return cc

            lax.fori_loop(g0, g1, grp, 0)
            rem = hi - lo

            @pl.when(rem == CH)
            def _():
                pltpu.async_copy(e_v, e_hbm.at[pl.ds(_m8(start * F), CH * F)], so)

            @pl.when((rem > 0) & (rem < CH))
            def _():
                for sz in (256, 128, 64, 32, 16, 8, 4, 2, 1):
                    off = lo + (rem // (2 * sz)) * (2 * sz)

                    @pl.when((rem // sz) % 2 == 1)
                    def _(off=off, sz=sz):
                        pltpu.sync_copy(
                            e_v.at[pl.ds(off * F, sz * F)],
                            e_hbm.at[pl.ds(_m8((start + off) * F), sz * F)])

        @pl.when(k % 2 == 0)
        def _():
            pltpu.make_async_copy(clslice(k), cl_v0, sc0).wait()
            proc(cl_v0, h_v0, so0)

        @pl.when(k % 2 == 1)
        def _():
            pltpu.make_async_copy(clslice(k), cl_v1, sc1).wait()
            proc(cl_v1, h_v1, so1)

        return c

    lax.fori_loop(0, nch, chunkB, 0)

    for d in (1, 2):
        kd = nch - d

        @pl.when((kd >= 0) & full(kd))
        def _(kd=kd):
            @pl.when(kd % 2 == 0)
            def _():
                pltpu.make_async_copy(
                    h_v0, e_hbm.at[pl.ds(_m8((start0 + kd * CH) * F), CH * F)],
                    so0).wait()

            @pl.when(kd % 2 == 1)
            def _():
                pltpu.make_async_copy(
                    h_v1, e_hbm.at[pl.ds(_m8((start0 + kd * CH) * F), CH * F)],
                    so1).wait()


@functools.partial(
    pl.kernel,
    out_type=[jax.ShapeDtypeStruct((NW * CT * F,), jnp.float32),
              jax.ShapeDtypeStruct((NW * CT,), jnp.int32)],
    mesh=_MESH,
    compiler_params=pltpu.CompilerParams(needs_layout_passes=False),
    scratch_types=[
        pltpu.VMEM((16,), jnp.int32),
        pltpu.VMEM((CT * F,), jnp.float32),
        pltpu.VMEM((CH * F,), jnp.float32),
        pltpu.VMEM((CH * F,), jnp.float32),
        pltpu.VMEM((CH + G,), jnp.int32),
        pltpu.VMEM((CH + G,), jnp.int32),
        pltpu.VMEM((CH + G,), jnp.int32),
        pltpu.VMEM((CH + G,), jnp.int32),
        pltpu.VMEM((CT,), jnp.int32),
        pltpu.SemaphoreType.DMA,
        pltpu.SemaphoreType.DMA,
        pltpu.SemaphoreType.DMA,
        pltpu.SemaphoreType.DMA,
        pltpu.SemaphoreType.DMA,
        pltpu.SemaphoreType.DMA,
    ],
)
def _segmax_batch(h_hbm, cl_hbm, bt_hbm, rs_hbm, aggr_hbm, pb_hbm,
                  rs_vm, aggr_v, h_v0, h_v1, cl_v0, cl_v1, bt_v0, bt_v1,
                  pb_vm, sh0, sh1, sc0, sc1, sb0, sb1):
    w = lax.axis_index("s") * NC + lax.axis_index("c")
    row_start, row_end = _load_bounds(rs_hbm, rs_vm, w)
    ct0 = w * CT

    _init_aggr(aggr_v)
    zero16 = jnp.zeros((16,), jnp.int32)

    def initpb(i, c):
        pb_vm[pl.ds(i * 16, 16)] = zero16
        return c

    lax.fori_loop(0, CT // 16, initpb, 0)

    start0 = (row_start // 8) * 8
    nch = (row_end - start0 + CH - 1) // CH

    def hslice(k):
        return h_hbm.at[pl.ds(_m8((start0 + k * CH) * F), CH * F)]

    def clslice(k):
        return cl_hbm.at[pl.ds(_m8(start0 + k * CH), CH + G)]

    def btslice(k):
        return bt_hbm.at[pl.ds(_m8(start0 + k * CH), CH + G)]

    @pl.when(nch > 0)
    def _():
        pltpu.async_copy(hslice(0), h_v0, sh0)
        pltpu.async_copy(clslice(0), cl_v0, sc0)
        pltpu.async_copy(btslice(0), bt_v0, sb0)

    def chunk(k, c):
        @pl.when(k + 1 < nch)
        def _():
            @pl.when(k % 2 == 0)
            def _():
                pltpu.async_copy(hslice(k + 1), h_v1, sh1)
                pltpu.async_copy(clslice(k + 1), cl_v1, sc1)
                pltpu.async_copy(btslice(k + 1), bt_v1, sb1)

            @pl.when(k % 2 == 1)
            def _():
                pltpu.async_copy(hslice(k + 1), h_v0, sh0)
                pltpu.async_copy(clslice(k + 1), cl_v0, sc0)
                pltpu.async_copy(btslice(k + 1), bt_v0, sb0)

        start = start0 + k * CH
        lo = jnp.maximum(row_start - start, 0)
        hi = jnp.minimum(row_end - start, CH)
        g0, g1 = _group_range(lo, hi)

        def proc(h_v, cl_vm, bt_vm):
            def grp(g, cc):
                _accum_group(cl_vm, h_v, aggr_v, g, lo, hi, ct0,
                             extra=(bt_vm, pb_vm))
                return cc

            lax.fori_loop(g0, g1, grp, 0)

        @pl.when(k % 2 == 0)
        def _():
            pltpu.make_async_copy(hslice(k), h_v0, sh0).wait()
            pltpu.make_async_copy(clslice(k), cl_v0, sc0).wait()
            pltpu.make_async_copy(btslice(k), bt_v0, sb0).wait()
            proc(h_v0, cl_v0, bt_v0)

        @pl.when(k % 2 == 1)
        def _():
            pltpu.make_async_copy(hslice(k), h_v1, sh1).wait()
            pltpu.make_async_copy(clslice(k), cl_v1, sc1).wait()
            pltpu.make_async_copy(btslice(k), bt_v1, sb1).wait()
            proc(h_v1, cl_v1, bt_v1)

        return c

    lax.fori_loop(0, nch, chunk, 0)

    _fix_neginf(aggr_v)
    pltpu.sync_copy(aggr_v, aggr_hbm.at[pl.ds(_m8(ct0 * F), CT * F)])
    pltpu.sync_copy(pb_vm, pb_hbm.at[pl.ds(_m8(ct0), CT)])


def kernel(x, cluster, batch, pre_W0, pre_b0, pre_W1, pre_b1,
           l1_W0, l1_b0, l1_g, l1_b, l1_W1, l1_b1,
           l2_W0, l2_b0, l2_g, l2_b, l2_W1, l2_b1,
           norm_g, norm_b):
    grid_a = (N_PAD // BLKA,)

    h1 = pl.pallas_call(
        _mlp1_body,
        grid=grid_a,
        in_specs=[
            pl.BlockSpec((BLKA, D_IN), lambda i: (i, 0)),
            _wspec((D_IN, H)), _wspec((1, H)),
            _wspec((H, H)), _wspec((1, H)),
            _wspec((H, H)), _wspec((1, H)),
            _wspec((1, H)), _wspec((1, H)),
            _wspec((H, F)), _wspec((1, F)),
        ],
        out_specs=pl.BlockSpec((BLKA, F), lambda i: (i, 0)),
        out_shape=jax.ShapeDtypeStruct((N_PAD, F), jnp.float32),
    )(x, pre_W0.astype(jnp.bfloat16), _row2(pre_b0),
      pre_W1.astype(jnp.bfloat16), _row2(pre_b1),
      l1_W0.astype(jnp.bfloat16), _row2(l1_b0), _row2(l1_g), _row2(l1_b),
      l1_W1.astype(jnp.bfloat16), _row2(l1_b1))

    bounds = jnp.arange(NW + 1, dtype=jnp.int32) * CT
    rs = jnp.searchsorted(cluster, bounds, side="left").astype(jnp.int32)
    rs2 = jnp.zeros((NW, 16), jnp.int32)
    rs2 = rs2.at[:, 0].set(rs[:NW]).at[:, 1].set(rs[1:NW + 1]).reshape(-1)
    cl_pad = jnp.pad(cluster, (0, N_PAD - N), constant_values=S)
    bt_pad = jnp.pad(batch, (0, N_PAD - N))

    h1f = h1.reshape(-1)
    h1p = h1f.reshape(N_PAD // 2, 2 * F)
    e1 = _segmax_expand(h1f, cl_pad, rs2).reshape(N_PAD // 2, 2 * F)

    wa = l2_W0[:F, :].astype(jnp.bfloat16)
    wb = l2_W0[F:, :].astype(jnp.bfloat16)
    zFH = jnp.zeros((F, H), jnp.bfloat16)
    wad = jnp.block([[wa, zFH], [zFH, wa]])
    wbd = jnp.block([[wb, zFH], [zFH, wb]])
    w21 = l2_W1.astype(jnp.bfloat16)
    zHF = jnp.zeros((H, F), jnp.bfloat16)
    w21d = jnp.block([[w21, zHF], [zHF, w21]])
    b20d = _row2(jnp.concatenate([l2_b0, l2_b0]))
    b21d = _row2(jnp.concatenate([l2_b1, l2_b1]))
    h2 = pl.pallas_call(
        _mlp2_body,
        grid=grid_a,
        in_specs=[
            pl.BlockSpec((BLKA // 2, 2 * F), lambda i: (i, 0)),
            pl.BlockSpec((BLKA // 2, 2 * F), lambda i: (i, 0)),
            _wspec((2 * F, 2 * H)), _wspec((2 * F, 2 * H)), _wspec((1, 2 * H)),
            _wspec((1, H)), _wspec((1, H)),
            _wspec((2 * H, 2 * F)), _wspec((1, 2 * F)),
        ],
        out_specs=pl.BlockSpec((BLKA // 2, 2 * F), lambda i: (i, 0)),
        out_shape=jax.ShapeDtypeStruct((N_PAD // 2, 2 * F), jnp.float32),
    )(h1p, e1, wad, wbd, b20d, _row2(l2_g), _row2(l2_b),
      w21d, b21d)

    aggr2, pb = _segmax_batch(h2.reshape(-1), cl_pad, bt_pad, rs2)
    aggr2 = aggr2.reshape(NW * CT, F)

    BLKE = 512
    grid_e = ((NW * CT) // BLKE,)
    out = pl.pallas_call(
        _final_body,
        grid=grid_e,
        in_specs=[
            pl.BlockSpec((BLKE, F), lambda i: (i, 0)),
            _wspec((1, H)), _wspec((1, H)),
        ],
        out_specs=pl.BlockSpec((BLKE, H), lambda i: (i, 0)),
        out_shape=jax.ShapeDtypeStruct((S, H), jnp.float32),
    )(aggr2, _row2(norm_g), _row2(norm_b))

    return out, pb[:S]


# speedup vs baseline: 3.7503x; 1.0142x over previous
import functools

import jax
import jax.numpy as jnp
from jax import lax
from jax.experimental import pallas as pl
from jax.experimental.pallas import tpu as pltpu
from jax.experimental.pallas import tpu_sc as plsc

N = 100000
S = 6250
D_IN = 128
H = 128
F = 64
FV = F // 16

NC = 2
NS = 16
NW = NC * NS
CT = 208
CH = 256

BLKA = 1024
N_PAD = ((N + CH + BLKA - 1) // BLKA) * BLKA


def _silu(x):
    return x * jax.nn.sigmoid(x)


def _ln(u, g, b):
    m = jnp.mean(u, axis=1, keepdims=True)
    ex2 = jnp.mean(u * u, axis=1, keepdims=True)
    v = ex2 - m * m
    return (u - m) * lax.rsqrt(v + 1e-5) * g + b


def _bdot(a, b):
    return jnp.dot(a.astype(jnp.bfloat16), b, preferred_element_type=jnp.float32)


def _mlp1_body(x_ref, w00, b00, wf, bf, g1, bb1, w11, b11, h_ref):
    t = _silu(_bdot(x_ref[...], w00[...]) + b00[...])
    u = jnp.dot(t, wf[...], preferred_element_type=jnp.float32) + bf[...]
    u = _silu(_ln(u, g1[...], bb1[...]))
    h_ref[...] = _bdot(u, w11[...]) + b11[...]


def _mlp2_body(h_ref, e_ref, wa, wb, b20, g2, bb2, w21, b21, h2_ref):
    u = _bdot(h_ref[...], wa[...]) + _bdot(e_ref[...], wb[...]) + b20[...]
    ul = _silu(_ln(u[:, :H], g2[...], bb2[...]))
    ur = _silu(_ln(u[:, H:], g2[...], bb2[...]))
    t = jnp.concatenate([ul, ur], axis=1)
    h2_ref[...] = _bdot(t, w21[...]) + b21[...]


def _final_body(a_ref, g_ref, b_ref, o_ref):
    a = a_ref[...]
    m = jnp.mean(a, axis=1, keepdims=True)
    v = jnp.mean((a - m) ** 2, axis=1, keepdims=True)
    nrm = (a - m) * lax.rsqrt(v + 1e-5)
    g = g_ref[...]
    b = b_ref[...]
    o_ref[...] = jnp.concatenate(
        [nrm * g[:, :F] + b[:, :F], nrm * g[:, F:] + b[:, F:]], axis=1)


def _row2(a):
    return a.reshape(1, -1)


def _wspec(shape):
    return pl.BlockSpec(shape, lambda i: (0, 0))


_MESH = plsc.VectorSubcoreMesh(core_axis_name="c", subcore_axis_name="s")
G = 16
NG = CH // G


def _m8(i):
    return pl.multiple_of(i, 8)


def _mn(i, n):
    return pl.multiple_of(i, n)


def _init_aggr(aggr_v, lanes=16, dtype=jnp.float32):
    neg = jnp.full((lanes,), -jnp.inf, dtype)

    def body(i, c):
        for j in range(F // lanes):
            aggr_v[pl.ds(i * F + j * lanes, lanes)] = neg
        return c

    lax.fori_loop(0, CT, body, 0)


def _fix_neginf(aggr_v, lanes=16, dtype=jnp.float32):
    neg = dtype(-jnp.inf)

    def body(i, c):
        for j in range(F // lanes):
            v = aggr_v[pl.ds(i * F + j * lanes, lanes)]
            aggr_v[pl.ds(i * F + j * lanes, lanes)] = jnp.where(
                v == neg, dtype(0.0), v)
        return c

    lax.fori_loop(0, CT, body, 0)


def _load_bounds(rs_hbm, rs_vm, w):
    pltpu.sync_copy(rs_hbm.at[pl.ds(_m8(w * 16), 16)], rs_vm)
    v = rs_vm[...]
    return v[0], v[1]


def _accum_group(cl_vm, h_v, aggr_v, g, lo, hi, ct0, lanes=16,
                 dtype=jnp.float32, extra=None):
    clv = cl_vm[pl.ds(g * G, G)]
    base = g * G
    neg = dtype(-jnp.inf)
    for k in range(G):
        r = base + k
        valid = (r >= lo) & (r < hi)
        ci = jnp.minimum(jnp.maximum(clv[k] - ct0, 0), CT - 1)
        for j in range(F // lanes):
            cur = aggr_v[pl.ds(ci * F + j * lanes, lanes)]
            hv = h_v[pl.ds(r * F + j * lanes, lanes)]
            hv = jnp.where(valid, hv, neg)
            aggr_v[pl.ds(ci * F + j * lanes, lanes)] = jnp.maximum(cur, hv)
    if extra is not None:
        bt_vm, pb_vm = extra
        nextv = cl_vm[pl.ds(g * G + 1, G)]
        btv = bt_vm[pl.ds(g * G, G)]
        rvec = base + lax.iota(jnp.int32, 16)
        vmask = (rvec >= lo) & (rvec < hi)
        emask = (clv != nextv) & vmask
        civ = jnp.minimum(jnp.maximum(clv - ct0, 0), CT - 1)
        plsc.store_scatter(pb_vm, [civ], btv, mask=emask)


def _group_range(lo, hi):
    return lo // G, (hi + G - 1) // G


@functools.partial(
    pl.kernel,
    out_type=jax.ShapeDtypeStruct((N_PAD * F,), jnp.float32),
    mesh=_MESH,
    compiler_params=pltpu.CompilerParams(needs_layout_passes=False),
    scratch_types=[
        pltpu.VMEM((16,), jnp.int32),
        pltpu.VMEM((CT * F,), jnp.float32),
        pltpu.VMEM((CH * F,), jnp.float32),
        pltpu.VMEM((CH * F,), jnp.float32),
        pltpu.VMEM((CH * F,), jnp.float32),
        pltpu.VMEM((CH * F,), jnp.float32),
        pltpu.VMEM((CH + G,), jnp.int32),
        pltpu.VMEM((CH + G,), jnp.int32),
        pltpu.SemaphoreType.DMA,
        pltpu.SemaphoreType.DMA,
        pltpu.SemaphoreType.DMA,
        pltpu.SemaphoreType.DMA,
        pltpu.SemaphoreType.DMA,
        pltpu.SemaphoreType.DMA,
    ],
)
def _segmax_expand(h_hbm, cl_hbm, rs_hbm, e_hbm,
                   rs_vm, aggr_v, h_v0, h_v1, e_v0, e_v1,
                   cl_v0, cl_v1, sh0, sh1, sc0, sc1, so0, so1):
    w = lax.axis_index("s") * NC + lax.axis_index("c")
    row_start, row_end = _load_bounds(rs_hbm, rs_vm, w)
    ct0 = w * CT
    _init_aggr(aggr_v)

    start0 = (row_start // 8) * 8
    nch = (row_end - start0 + CH - 1) // CH

    def hslice(k):
        return h_hbm.at[pl.ds(_m8((start0 + k * CH) * F), CH * F)]

    def clslice(k):
        return cl_hbm.at[pl.ds(_m8(start0 + k * CH), CH + G)]

    def lohi(k):
        start = start0 + k * CH
        return (jnp.maximum(row_start - start, 0),
                jnp.minimum(row_end - start, CH))

    @pl.when(nch > 0)
    def _():
        pltpu.async_copy(hslice(0), h_v0, sh0)
        pltpu.async_copy(clslice(0), cl_v0, sc0)

    def chunkA(k, c):
        @pl.when(k + 1 < nch)
        def _():
            @pl.when(k % 2 == 0)
            def _():
                pltpu.async_copy(hslice(k + 1), h_v1, sh1)
                pltpu.async_copy(clslice(k + 1), cl_v1, sc1)

            @pl.when(k % 2 == 1)
            def _():
                pltpu.async_copy(hslice(k + 1), h_v0, sh0)
                pltpu.async_copy(clslice(k + 1), cl_v0, sc0)

        lo, hi = lohi(k)
        g0, g1 = _group_range(lo, hi)

        def proc(h_v, cl_vm):
            def grp(g, cc):
                _accum_group(cl_vm, h_v, aggr_v, g, lo, hi, ct0)
                return cc

            lax.fori_loop(g0, g1, grp, 0)

        @pl.when(k % 2 == 0)
        def _():
            pltpu.make_async_copy(hslice(k), h_v0, sh0).wait()
            pltpu.make_async_copy(clslice(k), cl_v0, sc0).wait()
            proc(h_v0, cl_v0)

        @pl.when(k % 2 == 1)
        def _():
            pltpu.make_async_copy(hslice(k), h_v1, sh1).wait()
            pltpu.make_async_copy(clslice(k), cl_v1, sc1).wait()
            proc(h_v1, cl_v1)

        return c

    lax.fori_loop(0, nch, chunkA, 0)
    _fix_neginf(aggr_v)

    def full(k):
        lo, hi = lohi(k)
        return (lo == 0) & (hi == CH)

    @pl.when(nch > 0)
    def _():
        pltpu.async_copy(clslice(0), cl_v0, sc0)

    def chunkB(k, c):
        @pl.when(k + 1 < nch)
        def _():
            @pl.when(k % 2 == 0)
            def _():
                pltpu.async_copy(clslice(k + 1), cl_v1, sc1)

            @pl.when(k % 2 == 1)
            def _():
                pltpu.async_copy(clslice(k + 1), cl_v0, sc0)

        start = start0 + k * CH
        lo, hi = lohi(k)
        g0, g1 = _group_range(lo, hi)

        def proc(cl_vm, e_v, so):
            @pl.when((k >= 2) & full(k - 2))
            def _():
                pltpu.make_async_copy(
                    e_v,
                    e_hbm.at[pl.ds(_m8((start0 + (k - 2) * CH) * F), CH * F)],
                    so).wait()

            def grp(g, cc):
                clv = cl_vm[pl.ds(g * G, G)]
                base = g * G
                for k16 in range(G):
                    r = base + k16
                    ci = jnp.minimum(jnp.maximum(clv[k16] - ct0, 0), CT - 1)
                    for j in range(FV):
                        e_v[pl.ds(r * F + j * 16, 16)] = aggr_v[pl.ds(ci * F + j * 16, 16)]
                return cc

            lax.fori_loop(g0, g1, grp, 0)
            rem = hi - lo

            @pl.when(rem == CH)
            def _():
                pltpu.async_copy(e_v, e_hbm.at[pl.ds(_m8(start * F), CH * F)], so)

            @pl.when((rem > 0) & (rem < CH))
            def _():
                for sz in (256, 128, 64, 32, 16, 8, 4, 2, 1):
                    off = lo + (rem // (2 * sz)) * (2 * sz)

                    @pl.when((rem // sz) % 2 == 1)
                    def _(off=off, sz=sz):
                        pltpu.sync_copy(
                            e_v.at[pl.ds(off * F, sz * F)],
                            e_hbm.at[pl.ds(_m8((start + off) * F), sz * F)])

        @pl.when(k % 2 == 0)
        def _():
            pltpu.make_async_copy(clslice(k), cl_v0, sc0).wait()
            proc(cl_v0, e_v0, so0)

        @pl.when(k % 2 == 1)
        def _():
            pltpu.make_async_copy(clslice(k), cl_v1, sc1).wait()
            proc(cl_v1, e_v1, so1)

        return c

    lax.fori_loop(0, nch, chunkB, 0)

    for d in (1, 2):
        kd = nch - d

        @pl.when((kd >= 0) & full(kd))
        def _(kd=kd):
            @pl.when(kd % 2 == 0)
            def _():
                pltpu.make_async_copy(
                    e_v0, e_hbm.at[pl.ds(_m8((start0 + kd * CH) * F), CH * F)],
                    so0).wait()

            @pl.when(kd % 2 == 1)
            def _():
                pltpu.make_async_copy(
                    e_v1, e_hbm.at[pl.ds(_m8((start0 + kd * CH) * F), CH * F)],
                    so1).wait()


@functools.partial(
    pl.kernel,
    out_type=[jax.ShapeDtypeStruct((NW * CT * F,), jnp.float32),
              jax.ShapeDtypeStruct((NW * CT,), jnp.int32)],
    mesh=_MESH,
    compiler_params=pltpu.CompilerParams(needs_layout_passes=False),
    scratch_types=[
        pltpu.VMEM((16,), jnp.int32),
        pltpu.VMEM((CT * F,), jnp.float32),
        pltpu.VMEM((CH * F,), jnp.float32),
        pltpu.VMEM((CH * F,), jnp.float32),
        pltpu.VMEM((CH + G,), jnp.int32),
        pltpu.VMEM((CH + G,), jnp.int32),
        pltpu.VMEM((CH + G,), jnp.int32),
        pltpu.VMEM((CH + G,), jnp.int32),
        pltpu.VMEM((CT,), jnp.int32),
        pltpu.SemaphoreType.DMA,
        pltpu.SemaphoreType.DMA,
        pltpu.SemaphoreType.DMA,
        pltpu.SemaphoreType.DMA,
        pltpu.SemaphoreType.DMA,
        pltpu.SemaphoreType.DMA,
    ],
)
def _segmax_batch(h_hbm, cl_hbm, bt_hbm, rs_hbm, aggr_hbm, pb_hbm,
                  rs_vm, aggr_v, h_v0, h_v1, cl_v0, cl_v1, bt_v0, bt_v1,
                  pb_vm, sh0, sh1, sc0, sc1, sb0, sb1):
    w = lax.axis_index("s") * NC + lax.axis_index("c")
    row_start, row_end = _load_bounds(rs_hbm, rs_vm, w)
    ct0 = w * CT

    _init_aggr(aggr_v)
    zero16 = jnp.zeros((16,), jnp.int32)

    def initpb(i, c):
        pb_vm[pl.ds(i * 16, 16)] = zero16
        return c

    lax.fori_loop(0, CT // 16, initpb, 0)

    start0 = (row_start // 8) * 8
    nch = (row_end - start0 + CH - 1) // CH

    def hslice(k):
        return h_hbm.at[pl.ds(_m8((start0 + k * CH) * F), CH * F)]

    def clslice(k):
        return cl_hbm.at[pl.ds(_m8(start0 + k * CH), CH + G)]

    def btslice(k):
        return bt_hbm.at[pl.ds(_m8(start0 + k * CH), CH + G)]

    @pl.when(nch > 0)
    def _():
        pltpu.async_copy(hslice(0), h_v0, sh0)
        pltpu.async_copy(clslice(0), cl_v0, sc0)
        pltpu.async_copy(btslice(0), bt_v0, sb0)

    def chunk(k, c):
        @pl.when(k + 1 < nch)
        def _():
            @pl.when(k % 2 == 0)
            def _():
                pltpu.async_copy(hslice(k + 1), h_v1, sh1)
                pltpu.async_copy(clslice(k + 1), cl_v1, sc1)
                pltpu.async_copy(btslice(k + 1), bt_v1, sb1)

            @pl.when(k % 2 == 1)
            def _():
                pltpu.async_copy(hslice(k + 1), h_v0, sh0)
                pltpu.async_copy(clslice(k + 1), cl_v0, sc0)
                pltpu.async_copy(btslice(k + 1), bt_v0, sb0)

        start = start0 + k * CH
        lo = jnp.maximum(row_start - start, 0)
        hi = jnp.minimum(row_end - start, CH)
        g0, g1 = _group_range(lo, hi)

        def proc(h_v, cl_vm, bt_vm):
            def grp(g, cc):
                _accum_group(cl_vm, h_v, aggr_v, g, lo, hi, ct0,
                             extra=(bt_vm, pb_vm))
                return cc

            lax.fori_loop(g0, g1, grp, 0)

        @pl.when(k % 2 == 0)
        def _():
            pltpu.make_async_copy(hslice(k), h_v0, sh0).wait()
            pltpu.make_async_copy(clslice(k), cl_v0, sc0).wait()
            pltpu.make_async_copy(btslice(k), bt_v0, sb0).wait()
            proc(h_v0, cl_v0, bt_v0)

        @pl.when(k % 2 == 1)
        def _():
            pltpu.make_async_copy(hslice(k), h_v1, sh1).wait()
            pltpu.make_async_copy(clslice(k), cl_v1, sc1).wait()
            pltpu.make_async_copy(btslice(k), bt_v1, sb1).wait()
            proc(h_v1, cl_v1, bt_v1)

        return c

    lax.fori_loop(0, nch, chunk, 0)

    _fix_neginf(aggr_v)
    pltpu.sync_copy(aggr_v, aggr_hbm.at[pl.ds(_m8(ct0 * F), CT * F)])
    pltpu.sync_copy(pb_vm, pb_hbm.at[pl.ds(_m8(ct0), CT)])


def kernel(x, cluster, batch, pre_W0, pre_b0, pre_W1, pre_b1,
           l1_W0, l1_b0, l1_g, l1_b, l1_W1, l1_b1,
           l2_W0, l2_b0, l2_g, l2_b, l2_W1, l2_b1,
           norm_g, norm_b):
    grid_a = (N_PAD // BLKA,)

    wf = pre_W1 @ l1_W0
    bf = pre_b1 @ l1_W0 + l1_b0

    h1 = pl.pallas_call(
        _mlp1_body,
        grid=grid_a,
        in_specs=[
            pl.BlockSpec((BLKA, D_IN), lambda i: (i, 0)),
            _wspec((D_IN, H)), _wspec((1, H)),
            _wspec((H, H)), _wspec((1, H)),
            _wspec((1, H)), _wspec((1, H)),
            _wspec((H, F)), _wspec((1, F)),
        ],
        out_specs=pl.BlockSpec((BLKA, F), lambda i: (i, 0)),
        out_shape=jax.ShapeDtypeStruct((N_PAD, F), jnp.float32),
    )(x, pre_W0.astype(jnp.bfloat16), _row2(pre_b0),
      wf, _row2(bf),
      _row2(l1_g), _row2(l1_b),
      l1_W1.astype(jnp.bfloat16), _row2(l1_b1))

    bounds = jnp.arange(NW + 1, dtype=jnp.int32) * CT
    rs = jnp.searchsorted(cluster, bounds, side="left").astype(jnp.int32)
    rs2 = jnp.zeros((NW, 16), jnp.int32)
    rs2 = rs2.at[:, 0].set(rs[:NW]).at[:, 1].set(rs[1:NW + 1]).reshape(-1)
    cl_pad = jnp.pad(cluster, (0, N_PAD - N), constant_values=S)
    bt_pad = jnp.pad(batch, (0, N_PAD - N))

    h1f = h1.reshape(-1)
    h1p = h1f.reshape(N_PAD // 2, 2 * F)
    e1 = _segmax_expand(h1f, cl_pad, rs2).reshape(N_PAD // 2, 2 * F)

    wa = l2_W0[:F, :].astype(jnp.bfloat16)
    wb = l2_W0[F:, :].astype(jnp.bfloat16)
    zFH = jnp.zeros((F, H), jnp.bfloat16)
    wad = jnp.block([[wa, zFH], [zFH, wa]])
    wbd = jnp.block([[wb, zFH], [zFH, wb]])
    w21 = l2_W1.astype(jnp.bfloat16)
    zHF = jnp.zeros((H, F), jnp.bfloat16)
    w21d = jnp.block([[w21, zHF], [zHF, w21]])
    b20d = _row2(jnp.concatenate([l2_b0, l2_b0]))
    b21d = _row2(jnp.concatenate([l2_b1, l2_b1]))
    h2 = pl.pallas_call(
        _mlp2_body,
        grid=grid_a,
        in_specs=[
            pl.BlockSpec((BLKA // 2, 2 * F), lambda i: (i, 0)),
            pl.BlockSpec((BLKA // 2, 2 * F), lambda i: (i, 0)),
            _wspec((2 * F, 2 * H)), _wspec((2 * F, 2 * H)), _wspec((1, 2 * H)),
            _wspec((1, H)), _wspec((1, H)),
            _wspec((2 * H, 2 * F)), _wspec((1, 2 * F)),
        ],
        out_specs=pl.BlockSpec((BLKA // 2, 2 * F), lambda i: (i, 0)),
        out_shape=jax.ShapeDtypeStruct((N_PAD // 2, 2 * F), jnp.float32),
    )(h1p, e1, wad, wbd, b20d, _row2(l2_g), _row2(l2_b),
      w21d, b21d)

    aggr2, pb = _segmax_batch(h2.reshape(-1), cl_pad, bt_pad, rs2)
    aggr2 = aggr2.reshape(NW * CT, F)

    BLKE = 512
    grid_e = ((NW * CT) // BLKE,)
    out = pl.pallas_call(
        _final_body,
        grid=grid_e,
        in_specs=[
            pl.BlockSpec((BLKE, F), lambda i: (i, 0)),
            _wspec((1, H)), _wspec((1, H)),
        ],
        out_specs=pl.BlockSpec((BLKE, H), lambda i: (i, 0)),
        out_shape=jax.ShapeDtypeStruct((S, H), jnp.float32),
    )(aggr2, _row2(norm_g), _row2(norm_b))

    return out, pb[:S]


# speedup vs baseline: 3.7548x; 1.0012x over previous
import functools

import jax
import jax.numpy as jnp
from jax import lax
from jax.experimental import pallas as pl
from jax.experimental.pallas import tpu as pltpu
from jax.experimental.pallas import tpu_sc as plsc

N = 100000
S = 6250
D_IN = 128
H = 128
F = 64
FV = F // 16

NC = 2
NS = 16
NW = NC * NS
CT = 208
CH = 256

BLKA = 1024
N_PAD = ((N + CH + BLKA - 1) // BLKA) * BLKA


def _silu(x):
    return x * jax.nn.sigmoid(x)


def _ln(u, g, b):
    m = jnp.mean(u, axis=1, keepdims=True)
    ex2 = jnp.mean(u * u, axis=1, keepdims=True)
    v = ex2 - m * m
    return (u - m) * lax.rsqrt(v + 1e-5) * g + b


def _bdot(a, b):
    return jnp.dot(a.astype(jnp.bfloat16), b, preferred_element_type=jnp.float32)


def _mlp1_body(x_ref, w00, b00, wf, bf, g1, bb1, w11, b11, h_ref):
    t = _silu(_bdot(x_ref[...], w00[...]) + b00[...])
    u = jnp.dot(t, wf[...], preferred_element_type=jnp.float32) + bf[...]
    u = _silu(_ln(u, g1[...], bb1[...]))
    h_ref[...] = _bdot(u, w11[...]) + b11[...]


def _mlp2_body(h_ref, e_ref, wa, wb, b20, g2, bb2, w21, b21, h2_ref):
    u = _bdot(h_ref[...], wa[...]) + _bdot(e_ref[...], wb[...]) + b20[...]
    ul = _silu(_ln(u[:, :H], g2[...], bb2[...]))
    ur = _silu(_ln(u[:, H:], g2[...], bb2[...]))
    t = jnp.concatenate([ul, ur], axis=1)
    h2_ref[...] = _bdot(t, w21[...]) + b21[...]


def _final_body(a_ref, g_ref, b_ref, o_ref):
    a = a_ref[...]
    m = jnp.mean(a, axis=1, keepdims=True)
    v = jnp.mean((a - m) ** 2, axis=1, keepdims=True)
    nrm = (a - m) * lax.rsqrt(v + 1e-5)
    g = g_ref[...]
    b = b_ref[...]
    o_ref[...] = jnp.concatenate(
        [nrm * g[:, :F] + b[:, :F], nrm * g[:, F:] + b[:, F:]], axis=1)


def _row2(a):
    return a.reshape(1, -1)


def _wspec(shape):
    return pl.BlockSpec(shape, lambda i: (0, 0))


_MESH = plsc.VectorSubcoreMesh(core_axis_name="c", subcore_axis_name="s")
G = 16
NG = CH // G


def _m8(i):
    return pl.multiple_of(i, 8)


def _mn(i, n):
    return pl.multiple_of(i, n)


def _init_aggr(aggr_v, lanes=16, dtype=jnp.float32):
    neg = jnp.full((lanes,), -jnp.inf, dtype)

    def body(i, c):
        for j in range(F // lanes):
            aggr_v[pl.ds(i * F + j * lanes, lanes)] = neg
        return c

    lax.fori_loop(0, CT, body, 0)


def _fix_neginf(aggr_v, lanes=16, dtype=jnp.float32):
    neg = dtype(-jnp.inf)

    def body(i, c):
        for j in range(F // lanes):
            v = aggr_v[pl.ds(i * F + j * lanes, lanes)]
            aggr_v[pl.ds(i * F + j * lanes, lanes)] = jnp.where(
                v == neg, dtype(0.0), v)
        return c

    lax.fori_loop(0, CT, body, 0)


def _load_bounds(rs_hbm, rs_vm, w):
    pltpu.sync_copy(rs_hbm.at[pl.ds(_m8(w * 16), 16)], rs_vm)
    v = rs_vm[...]
    return v[0], v[1]


def _accum_group(cl_vm, h_v, aggr_v, g, lo, hi, ct0, lanes=16,
                 dtype=jnp.float32, extra=None):
    clv = cl_vm[pl.ds(g * G, G)]
    base = g * G
    neg = dtype(-jnp.inf)
    for k in range(G):
        r = base + k
        valid = (r >= lo) & (r < hi)
        ci = jnp.minimum(jnp.maximum(clv[k] - ct0, 0), CT - 1)
        for j in range(F // lanes):
            cur = aggr_v[pl.ds(ci * F + j * lanes, lanes)]
            hv = h_v[pl.ds(r * F + j * lanes, lanes)]
            hv = jnp.where(valid, hv, neg)
            aggr_v[pl.ds(ci * F + j * lanes, lanes)] = jnp.maximum(cur, hv)
    if extra is not None:
        bt_vm, pb_vm = extra
        nextv = cl_vm[pl.ds(g * G + 1, G)]
        btv = bt_vm[pl.ds(g * G, G)]
        rvec = base + lax.iota(jnp.int32, 16)
        vmask = (rvec >= lo) & (rvec < hi)
        emask = (clv != nextv) & vmask
        civ = jnp.minimum(jnp.maximum(clv - ct0, 0), CT - 1)
        plsc.store_scatter(pb_vm, [civ], btv, mask=emask)


def _group_range(lo, hi):
    return lo // G, (hi + G - 1) // G


@functools.partial(
    pl.kernel,
    out_type=jax.ShapeDtypeStruct((N_PAD * F,), jnp.float32),
    mesh=_MESH,
    compiler_params=pltpu.CompilerParams(needs_layout_passes=False),
    scratch_types=[
        pltpu.VMEM((16,), jnp.int32),
        pltpu.VMEM((CT * F,), jnp.float32),
        pltpu.VMEM((CH * F,), jnp.float32),
        pltpu.VMEM((CH * F,), jnp.float32),
        pltpu.VMEM((CH * F,), jnp.float32),
        pltpu.VMEM((CH * F,), jnp.float32),
        pltpu.VMEM((CH + G,), jnp.int32),
        pltpu.VMEM((CH + G,), jnp.int32),
        pltpu.SemaphoreType.DMA,
        pltpu.SemaphoreType.DMA,
        pltpu.SemaphoreType.DMA,
        pltpu.SemaphoreType.DMA,
        pltpu.SemaphoreType.DMA,
        pltpu.SemaphoreType.DMA,
    ],
)
def _segmax_expand(h_hbm, cl_hbm, rs_hbm, e_hbm,
                   rs_vm, aggr_v, h_v0, h_v1, e_v0, e_v1,
                   cl_v0, cl_v1, sh0, sh1, sc0, sc1, so0, so1):
    w = lax.axis_index("s") * NC + lax.axis_index("c")
    row_start, row_end = _load_bounds(rs_hbm, rs_vm, w)
    ct0 = w * CT
    _init_aggr(aggr_v)

    start0 = (row_start // 8) * 8
    nch = (row_end - start0 + CH - 1) // CH

    def hslice(k):
        return h_hbm.at[pl.ds(_m8((start0 + k * CH) * F), CH * F)]

    def clslice(k):
        return cl_hbm.at[pl.ds(_m8(start0 + k * CH), CH + G)]

    def lohi(k):
        start = start0 + k * CH
        return (jnp.maximum(row_start - start, 0),
                jnp.minimum(row_end - start, CH))

    @pl.when(nch > 0)
    def _():
        pltpu.async_copy(hslice(0), h_v0, sh0)
        pltpu.async_copy(clslice(0), cl_v0, sc0)

    def chunkA(k, c):
        @pl.when(k + 1 < nch)
        def _():
            @pl.when(k % 2 == 0)
            def _():
                pltpu.async_copy(hslice(k + 1), h_v1, sh1)
                pltpu.async_copy(clslice(k + 1), cl_v1, sc1)

            @pl.when(k % 2 == 1)
            def _():
                pltpu.async_copy(hslice(k + 1), h_v0, sh0)
                pltpu.async_copy(clslice(k + 1), cl_v0, sc0)

        lo, hi = lohi(k)
        g0, g1 = _group_range(lo, hi)

        def proc(h_v, cl_vm):
            def grp(g, cc):
                _accum_group(cl_vm, h_v, aggr_v, g, lo, hi, ct0)
                return cc

            lax.fori_loop(g0, g1, grp, 0)

        @pl.when(k % 2 == 0)
        def _():
            pltpu.make_async_copy(hslice(k), h_v0, sh0).wait()
            pltpu.make_async_copy(clslice(k), cl_v0, sc0).wait()
            proc(h_v0, cl_v0)

        @pl.when(k % 2 == 1)
        def _():
            pltpu.make_async_copy(hslice(k), h_v1, sh1).wait()
            pltpu.make_async_copy(clslice(k), cl_v1, sc1).wait()
            proc(h_v1, cl_v1)

        return c

    lax.fori_loop(0, nch, chunkA, 0)
    _fix_neginf(aggr_v)

    def full(k):
        lo, hi = lohi(k)
        return (lo == 0) & (hi == CH)

    @pl.when(nch > 0)
    def _():
        pltpu.async_copy(clslice(0), cl_v0, sc0)

    def chunkB(k, c):
        @pl.when(k + 1 < nch)
        def _():
            @pl.when(k % 2 == 0)
            def _():
                pltpu.async_copy(clslice(k + 1), cl_v1, sc1)

            @pl.when(k % 2 == 1)
            def _():
                pltpu.async_copy(clslice(k + 1), cl_v0, sc0)

        start = start0 + k * CH
        lo, hi = lohi(k)
        g0, g1 = _group_range(lo, hi)

        def proc(cl_vm, e_v, so):
            @pl.when((k >= 2) & full(k - 2))
            def _():
                pltpu.make_async_copy(
                    e_v,
                    e_hbm.at[pl.ds(_m8((start0 + (k - 2) * CH) * F), CH * F)],
                    so).wait()

            def grp(g, cc):
                clv = cl_vm[pl.ds(g * G, G)]
                base = g * G
                for k16 in range(G):
                    r = base + k16
                    ci = jnp.minimum(jnp.maximum(clv[k16] - ct0, 0), CT - 1)
                    for j in range(FV):
                        e_v[pl.ds(r * F + j * 16, 16)] = aggr_v[pl.ds(ci * F + j * 16, 16)]
                return cc

            lax.fori_loop(g0, g1, grp, 0)
            rem = hi - lo

            @pl.when(rem == CH)
            def _():
                pltpu.async_copy(e_v, e_hbm.at[pl.ds(_m8(start * F), CH * F)], so)

            @pl.when((rem > 0) & (rem < CH))
            def _():
                for sz in (256, 128, 64, 32, 16, 8, 4, 2, 1):
                    off = lo + (rem // (2 * sz)) * (2 * sz)

                    @pl.when((rem // sz) % 2 == 1)
                    def _(off=off, sz=sz):
                        pltpu.sync_copy(
                            e_v.at[pl.ds(off * F, sz * F)],
                            e_hbm.at[pl.ds(_m8((start + off) * F), sz * F)])

        @pl.when(k % 2 == 0)
        def _():
            pltpu.make_async_copy(clslice(k), cl_v0, sc0).wait()
            proc(cl_v0, e_v0, so0)

        @pl.when(k % 2 == 1)
        def _():
            pltpu.make_async_copy(clslice(k), cl_v1, sc1).wait()
            proc(cl_v1, e_v1, so1)

        return c

    lax.fori_loop(0, nch, chunkB, 0)

    for d in (1, 2):
        kd = nch - d

        @pl.when((kd >= 0) & full(kd))
        def _(kd=kd):
            @pl.when(kd % 2 == 0)
            def _():
                pltpu.make_async_copy(
                    e_v0, e_hbm.at[pl.ds(_m8((start0 + kd * CH) * F), CH * F)],
                    so0).wait()

            @pl.when(kd % 2 == 1)
            def _():
                pltpu.make_async_copy(
                    e_v1, e_hbm.at[pl.ds(_m8((start0 + kd * CH) * F), CH * F)],
                    so1).wait()


@functools.partial(
    pl.kernel,
    out_type=[jax.ShapeDtypeStruct((NW * CT * F,), jnp.float32),
              jax.ShapeDtypeStruct((NW * CT,), jnp.int32)],
    mesh=_MESH,
    compiler_params=pltpu.CompilerParams(needs_layout_passes=False),
    scratch_types=[
        pltpu.VMEM((16,), jnp.int32),
        pltpu.VMEM((CT * F,), jnp.float32),
        pltpu.VMEM((CH * F,), jnp.float32),
        pltpu.VMEM((CH * F,), jnp.float32),
        pltpu.VMEM((CH + G,), jnp.int32),
        pltpu.VMEM((CH + G,), jnp.int32),
        pltpu.VMEM((CH + G,), jnp.int32),
        pltpu.VMEM((CH + G,), jnp.int32),
        pltpu.VMEM((CT,), jnp.int32),
        pltpu.SemaphoreType.DMA,
        pltpu.SemaphoreType.DMA,
        pltpu.SemaphoreType.DMA,
        pltpu.SemaphoreType.DMA,
        pltpu.SemaphoreType.DMA,
        pltpu.SemaphoreType.DMA,
    ],
)
def _segmax_batch(h_hbm, cl_hbm, bt_hbm, rs_hbm, aggr_hbm, pb_hbm,
                  rs_vm, aggr_v, h_v0, h_v1, cl_v0, cl_v1, bt_v0, bt_v1,
                  pb_vm, sh0, sh1, sc0, sc1, sb0, sb1):
    w = lax.axis_index("s") * NC + lax.axis_index("c")
    row_start, row_end = _load_bounds(rs_hbm, rs_vm, w)
    ct0 = w * CT

    _init_aggr(aggr_v)
    zero16 = jnp.zeros((16,), jnp.int32)

    def initpb(i, c):
        pb_vm[pl.ds(i * 16, 16)] = zero16
        return c

    lax.fori_loop(0, CT // 16, initpb, 0)

    start0 = (row_start // 8) * 8
    nch = (row_end - start0 + CH - 1) // CH

    def hslice(k):
        return h_hbm.at[pl.ds(_m8((start0 + k * CH) * F), CH * F)]

    def clslice(k):
        return cl_hbm.at[pl.ds(_m8(start0 + k * CH), CH + G)]

    def btslice(k):
        return bt_hbm.at[pl.ds(_m8(start0 + k * CH), CH + G)]

    @pl.when(nch > 0)
    def _():
        pltpu.async_copy(hslice(0), h_v0, sh0)
        pltpu.async_copy(clslice(0), cl_v0, sc0)
        pltpu.async_copy(btslice(0), bt_v0, sb0)

    def chunk(k, c):
        @pl.when(k + 1 < nch)
        def _():
            @pl.when(k % 2 == 0)
            def _():
                pltpu.async_copy(hslice(k + 1), h_v1, sh1)
                pltpu.async_copy(clslice(k + 1), cl_v1, sc1)
                pltpu.async_copy(btslice(k + 1), bt_v1, sb1)

            @pl.when(k % 2 == 1)
            def _():
                pltpu.async_copy(hslice(k + 1), h_v0, sh0)
                pltpu.async_copy(clslice(k + 1), cl_v0, sc0)
                pltpu.async_copy(btslice(k + 1), bt_v0, sb0)

        start = start0 + k * CH
        lo = jnp.maximum(row_start - start, 0)
        hi = jnp.minimum(row_end - start, CH)
        g0, g1 = _group_range(lo, hi)

        def proc(h_v, cl_vm, bt_vm):
            def grp(g, cc):
                _accum_group(cl_vm, h_v, aggr_v, g, lo, hi, ct0,
                             extra=(bt_vm, pb_vm))
                return cc

            lax.fori_loop(g0, g1, grp, 0)

        @pl.when(k % 2 == 0)
        def _():
            pltpu.make_async_copy(hslice(k), h_v0, sh0).wait()
            pltpu.make_async_copy(clslice(k), cl_v0, sc0).wait()
            pltpu.make_async_copy(btslice(k), bt_v0, sb0).wait()
            proc(h_v0, cl_v0, bt_v0)

        @pl.when(k % 2 == 1)
        def _():
            pltpu.make_async_copy(hslice(k), h_v1, sh1).wait()
            pltpu.make_async_copy(clslice(k), cl_v1, sc1).wait()
            pltpu.make_async_copy(btslice(k), bt_v1, sb1).wait()
            proc(h_v1, cl_v1, bt_v1)

        return c

    lax.fori_loop(0, nch, chunk, 0)

    _fix_neginf(aggr_v)
    pltpu.sync_copy(aggr_v, aggr_hbm.at[pl.ds(_m8(ct0 * F), CT * F)])
    pltpu.sync_copy(pb_vm, pb_hbm.at[pl.ds(_m8(ct0), CT)])


def kernel(x, cluster, batch, pre_W0, pre_b0, pre_W1, pre_b1,
           l1_W0, l1_b0, l1_g, l1_b, l1_W1, l1_b1,
           l2_W0, l2_b0, l2_g, l2_b, l2_W1, l2_b1,
           norm_g, norm_b):
    grid_a = (N_PAD // BLKA,)

    wf = pre_W1 @ l1_W0
    bf = pre_b1 @ l1_W0 + l1_b0

    h1 = pl.pallas_call(
        _mlp1_body,
        grid=grid_a,
        in_specs=[
            pl.BlockSpec((BLKA, D_IN), lambda i: (i, 0)),
            _wspec((D_IN, H)), _wspec((1, H)),
            _wspec((H, H)), _wspec((1, H)),
            _wspec((1, H)), _wspec((1, H)),
            _wspec((H, F)), _wspec((1, F)),
        ],
        out_specs=pl.BlockSpec((BLKA, F), lambda i: (i, 0)),
        out_shape=jax.ShapeDtypeStruct((N_PAD, F), jnp.float32),
    )(x, pre_W0.astype(jnp.bfloat16), _row2(pre_b0),
      wf, _row2(bf),
      _row2(l1_g), _row2(l1_b),
      l1_W1.astype(jnp.bfloat16), _row2(l1_b1))

    bounds = jnp.arange(NW + 1, dtype=jnp.int32) * CT
    rs = jnp.searchsorted(cluster, bounds, side="left").astype(jnp.int32)
    rs2 = jnp.zeros((NW, 16), jnp.int32)
    rs2 = rs2.at[:, 0].set(rs[:NW]).at[:, 1].set(rs[1:NW + 1]).reshape(-1)
    cl_pad = cluster
    bt_pad = batch

    h1f = h1.reshape(-1)
    h1p = h1f.reshape(N_PAD // 2, 2 * F)
    e1 = _segmax_expand(h1f, cl_pad, rs2).reshape(N_PAD // 2, 2 * F)

    wa = l2_W0[:F, :].astype(jnp.bfloat16)
    wb = l2_W0[F:, :].astype(jnp.bfloat16)
    zFH = jnp.zeros((F, H), jnp.bfloat16)
    wad = jnp.block([[wa, zFH], [zFH, wa]])
    wbd = jnp.block([[wb, zFH], [zFH, wb]])
    w21 = l2_W1.astype(jnp.bfloat16)
    zHF = jnp.zeros((H, F), jnp.bfloat16)
    w21d = jnp.block([[w21, zHF], [zHF, w21]])
    b20d = _row2(jnp.concatenate([l2_b0, l2_b0]))
    b21d = _row2(jnp.concatenate([l2_b1, l2_b1]))
    h2 = pl.pallas_call(
        _mlp2_body,
        grid=grid_a,
        in_specs=[
            pl.BlockSpec((BLKA // 2, 2 * F), lambda i: (i, 0)),
            pl.BlockSpec((BLKA // 2, 2 * F), lambda i: (i, 0)),
            _wspec((2 * F, 2 * H)), _wspec((2 * F, 2 * H)), _wspec((1, 2 * H)),
            _wspec((1, H)), _wspec((1, H)),
            _wspec((2 * H, 2 * F)), _wspec((1, 2 * F)),
        ],
        out_specs=pl.BlockSpec((BLKA // 2, 2 * F), lambda i: (i, 0)),
        out_shape=jax.ShapeDtypeStruct((N_PAD // 2, 2 * F), jnp.float32),
    )(h1p, e1, wad, wbd, b20d, _row2(l2_g), _row2(l2_b),
      w21d, b21d)

    aggr2, pb = _segmax_batch(h2.reshape(-1), cl_pad, bt_pad, rs2)
    aggr2 = aggr2.reshape(NW * CT, F)

    BLKE = 512
    grid_e = ((NW * CT) // BLKE,)
    out = pl.pallas_call(
        _final_body,
        grid=grid_e,
        in_specs=[
            pl.BlockSpec((BLKE, F), lambda i: (i, 0)),
            _wspec((1, H)), _wspec((1, H)),
        ],
        out_specs=pl.BlockSpec((BLKE, H), lambda i: (i, 0)),
        out_shape=jax.ShapeDtypeStruct((S, H), jnp.float32),
    )(aggr2, _row2(norm_g), _row2(norm_b))

    return out, pb[:S]
